# baseline (speedup 1.0000x reference)
import functools

import jax
import jax.numpy as jnp
from jax import lax
from jax.experimental import pallas as pl
from jax.experimental.pallas import tpu as pltpu
from jax.experimental.pallas import tpu_sc as plsc


def _xw_body(x_ref, w_ref, out_ref):
    out_ref[0] = jnp.dot(x_ref[...], w_ref[0],
                         preferred_element_type=jnp.float32)


def _compute_xw(x, weight, bn):
    n, d = x.shape
    r = weight.shape[0]
    return pl.pallas_call(
        _xw_body,
        grid=(n // bn, r),
        in_specs=[
            pl.BlockSpec((bn, d), lambda i, ri: (i, 0)),
            pl.BlockSpec((1, d, d), lambda i, ri: (ri, 0, 0)),
        ],
        out_specs=pl.BlockSpec((1, bn, d), lambda i, ri: (ri, i, 0)),
        out_shape=jax.ShapeDtypeStruct((r, n, d), jnp.float32),
    )(x, weight)


def _sc_gather_scatter(xw_flat, src_f, et_f, dst_f, zeros_blk, npad, d,
                       c_sz, nt, sb_sz, n):
    nsb = nt // sb_sz
    nc = sb_sz // c_sz
    rows = npad // 16
    mesh = plsc.VectorSubcoreMesh(core_axis_name="c", subcore_axis_name="s")

    @functools.partial(
        pl.kernel,
        mesh=mesh,
        out_type=jax.ShapeDtypeStruct((2, npad, d), jnp.float32),
        scratch_types=[
            pltpu.VMEM((sb_sz,), jnp.int32),
            pltpu.VMEM((sb_sz,), jnp.int32),
            pltpu.VMEM((sb_sz,), jnp.int32),
            pltpu.VMEM((c_sz, d), jnp.float32),
            pltpu.VMEM((c_sz, d), jnp.float32),
            pltpu.VMEM_SHARED((npad, d), jnp.float32),
            pltpu.SemaphoreType.DMA,
            pltpu.SemaphoreType.DMA,
        ],
    )
    def sc_kernel(xw_hbm, src_hbm, et_hbm, dst_hbm, zeros_hbm, out_hbm,
                  fidx_v, et_v, dst_v, rows_a, rows_b, acc_sh, sem_a,
                  sem_b):
        core = lax.axis_index("c")
        sub = lax.axis_index("s")
        wid = core * 16 + sub
        pltpu.sync_copy(zeros_hbm, acc_sh.at[pl.ds(sub * rows, rows)])
        plsc.subcore_barrier()

        def start_gather(i, buf, sem):
            i = jnp.minimum(i, nc - 1)
            off = pl.multiple_of(i * c_sz, 8)
            return pltpu.async_copy(
                xw_hbm.at[fidx_v.at[pl.ds(off, c_sz)]], buf, sem)

        def wait_gather(buf, sem):
            pltpu.make_async_copy(
                xw_hbm.at[fidx_v.at[pl.ds(0, c_sz)]], buf, sem).wait()

        def superblock(sb, carry):
            base = pl.multiple_of(wid * nt + sb * sb_sz, 8)
            pltpu.sync_copy(src_hbm.at[pl.ds(base, sb_sz)], fidx_v)
            pltpu.sync_copy(et_hbm.at[pl.ds(base, sb_sz)], et_v)
            pltpu.sync_copy(dst_hbm.at[pl.ds(base, sb_sz)], dst_v)

            def fx(g, cf):
                goff = pl.multiple_of(g * 16, 16)
                fidx_v[pl.ds(goff, 16)] = (et_v[pl.ds(goff, 16)] * n
                                           + fidx_v[pl.ds(goff, 16)])
                return cf

            lax.fori_loop(0, sb_sz // 16, fx, 0)
            start_gather(0, rows_a, sem_a)

            def dst_slice(i):
                off = pl.multiple_of(i * c_sz, 8)
                return dst_v.at[pl.ds(off, c_sz)]

            def pair(j, carry2):
                i0 = j * 2
                start_gather(i0 + 1, rows_b, sem_b)
                wait_gather(rows_a, sem_a)
                pltpu.sync_copy(rows_a, acc_sh.at[dst_slice(i0)], add=True)
                start_gather(i0 + 2, rows_a, sem_a)
                wait_gather(rows_b, sem_b)
                pltpu.sync_copy(rows_b, acc_sh.at[dst_slice(i0 + 1)],
                                add=True)
                return carry2

            lax.fori_loop(0, nc // 2, pair, 0)
            wait_gather(rows_a, sem_a)
            return carry

        lax.fori_loop(0, nsb, superblock, 0)
        plsc.subcore_barrier()
        pltpu.sync_copy(acc_sh.at[pl.ds(sub * rows, rows)],
                        out_hbm.at[core, pl.ds(sub * rows, rows)])

    return sc_kernel(xw_flat, src_f, et_f, dst_f, zeros_blk)


def _selfp_body(x_ref, w_ref, b_ref, out_ref):
    out_ref[...] = (jnp.dot(x_ref[...], w_ref[...],
                            preferred_element_type=jnp.float32)
                    + b_ref[...])


def _selfp(x, w_self, bias, bn):
    n, d = x.shape
    return pl.pallas_call(
        _selfp_body,
        grid=(n // bn,),
        in_specs=[
            pl.BlockSpec((bn, d), lambda i: (i, 0)),
            pl.BlockSpec((d, d), lambda i: (0, 0)),
            pl.BlockSpec((1, d), lambda i: (0, 0)),
        ],
        out_specs=pl.BlockSpec((bn, d), lambda i: (i, 0)),
        out_shape=jax.ShapeDtypeStruct((n, d), jnp.float32),
    )(x, w_self, bias.reshape(1, d))


def _final_body(acc_ref, sp_ref, out_ref):
    t = acc_ref[0] + acc_ref[1] + sp_ref[...]
    out_ref[...] = jnp.maximum(t, 0.0)


def _final(acc, selfp, bn):
    n, d = selfp.shape
    return pl.pallas_call(
        _final_body,
        grid=(n // bn,),
        in_specs=[
            pl.BlockSpec((2, bn, d), lambda i: (0, i, 0)),
            pl.BlockSpec((bn, d), lambda i: (i, 0)),
        ],
        out_specs=pl.BlockSpec((bn, d), lambda i: (i, 0)),
        out_shape=jax.ShapeDtypeStruct((n, d), jnp.float32),
    )(acc, selfp)


def kernel(x, edge_index, edge_type, num_entities, weight,
           self_loop_weight, bias):
    n, d = x.shape
    r = weight.shape[0]
    e = edge_type.shape[0]
    src = edge_index[0]
    dst = edge_index[1]

    xw = _compute_xw(x, weight, bn=2000)
    xw_flat = xw.reshape(r * n, d)

    nw = 32
    nt = e // nw
    sb_sz = 2000
    nsb = nt // sb_sz
    c_sz = 40
    nc = sb_sz // c_sz
    npad = ((n // 16 + 7) // 8 * 8) * 16
    zeros_blk = jnp.zeros((npad // 16, d), jnp.float32)

    acc = _sc_gather_scatter(xw_flat, src, edge_type, dst, zeros_blk,
                             npad, d, c_sz, nt, sb_sz, n)
    selfp = _selfp(x, self_loop_weight, bias, bn=2000)
    return _final(acc, selfp, bn=1000)


# speedup vs baseline: 1.0075x; 1.0075x over previous
import functools

import jax
import jax.numpy as jnp
from jax import lax
from jax.experimental import pallas as pl
from jax.experimental.pallas import tpu as pltpu
from jax.experimental.pallas import tpu_sc as plsc


def _xw_body(x_ref, w_ref, out_ref):
    out_ref[0] = jnp.dot(x_ref[...], w_ref[0],
                         preferred_element_type=jnp.float32)


def _compute_xw(x, weight, bn):
    n, d = x.shape
    r = weight.shape[0]
    return pl.pallas_call(
        _xw_body,
        grid=(n // bn, r),
        in_specs=[
            pl.BlockSpec((bn, d), lambda i, ri: (i, 0)),
            pl.BlockSpec((1, d, d), lambda i, ri: (ri, 0, 0)),
        ],
        out_specs=pl.BlockSpec((1, bn, d), lambda i, ri: (ri, i, 0)),
        out_shape=jax.ShapeDtypeStruct((r, n, d), jnp.float32),
    )(x, weight)


def _fidx_body(n, src_ref, et_ref, out_ref):
    out_ref[...] = et_ref[...] * n + src_ref[...]


def _compute_fidx(src, edge_type, n):
    e = src.shape[0]
    src2 = src.reshape(e // 128, 128)
    et2 = edge_type.reshape(e // 128, 128)
    out = pl.pallas_call(
        functools.partial(_fidx_body, n),
        out_shape=jax.ShapeDtypeStruct((e // 128, 128), jnp.int32),
    )(src2, et2)
    return out.reshape(e)


def _sc_gather_scatter(xw_flat, fidx_f, dst_f, zeros_blk, npad, d, c_sz,
                       nt, sb_sz):
    nsb = nt // sb_sz
    nc = sb_sz // c_sz
    rows = npad // 16
    mesh = plsc.VectorSubcoreMesh(core_axis_name="c", subcore_axis_name="s")

    @functools.partial(
        pl.kernel,
        mesh=mesh,
        out_type=jax.ShapeDtypeStruct((2, npad, d), jnp.float32),
        scratch_types=[
            pltpu.VMEM((sb_sz,), jnp.int32),
            pltpu.VMEM((sb_sz,), jnp.int32),
            pltpu.VMEM((c_sz, d), jnp.float32),
            pltpu.VMEM((c_sz, d), jnp.float32),
            pltpu.VMEM_SHARED((npad, d), jnp.float32),
            pltpu.SemaphoreType.DMA,
            pltpu.SemaphoreType.DMA,
        ],
    )
    def sc_kernel(xw_hbm, fidx_hbm, dst_hbm, zeros_hbm, out_hbm,
                  fidx_v, dst_v, rows_a, rows_b, acc_sh, sem_a, sem_b):
        core = lax.axis_index("c")
        sub = lax.axis_index("s")
        wid = core * 16 + sub
        pltpu.sync_copy(zeros_hbm, acc_sh.at[pl.ds(sub * rows, rows)])
        plsc.subcore_barrier()

        def start_gather(i, buf, sem):
            i = jnp.minimum(i, nc - 1)
            off = pl.multiple_of(i * c_sz, 8)
            return pltpu.async_copy(
                xw_hbm.at[fidx_v.at[pl.ds(off, c_sz)]], buf, sem)

        def wait_gather(buf, sem):
            pltpu.make_async_copy(
                xw_hbm.at[fidx_v.at[pl.ds(0, c_sz)]], buf, sem).wait()

        def superblock(sb, carry):
            base = pl.multiple_of(wid * nt + sb * sb_sz, 8)
            pltpu.sync_copy(fidx_hbm.at[pl.ds(base, sb_sz)], fidx_v)
            pltpu.sync_copy(dst_hbm.at[pl.ds(base, sb_sz)], dst_v)
            start_gather(0, rows_a, sem_a)

            def dst_slice(i):
                off = pl.multiple_of(i * c_sz, 8)
                return dst_v.at[pl.ds(off, c_sz)]

            def pair(j, carry2):
                i0 = j * 2
                start_gather(i0 + 1, rows_b, sem_b)
                wait_gather(rows_a, sem_a)
                pltpu.sync_copy(rows_a, acc_sh.at[dst_slice(i0)], add=True)
                start_gather(i0 + 2, rows_a, sem_a)
                wait_gather(rows_b, sem_b)
                pltpu.sync_copy(rows_b, acc_sh.at[dst_slice(i0 + 1)],
                                add=True)
                return carry2

            lax.fori_loop(0, nc // 2, pair, 0)
            wait_gather(rows_a, sem_a)
            return carry

        lax.fori_loop(0, nsb, superblock, 0)
        plsc.subcore_barrier()
        pltpu.sync_copy(acc_sh.at[pl.ds(sub * rows, rows)],
                        out_hbm.at[core, pl.ds(sub * rows, rows)])

    return sc_kernel(xw_flat, fidx_f, dst_f, zeros_blk)


def _selfp_body(x_ref, w_ref, b_ref, out_ref):
    out_ref[...] = (jnp.dot(x_ref[...], w_ref[...],
                            preferred_element_type=jnp.float32)
                    + b_ref[...])


def _selfp(x, w_self, bias, bn):
    n, d = x.shape
    return pl.pallas_call(
        _selfp_body,
        grid=(n // bn,),
        in_specs=[
            pl.BlockSpec((bn, d), lambda i: (i, 0)),
            pl.BlockSpec((d, d), lambda i: (0, 0)),
            pl.BlockSpec((1, d), lambda i: (0, 0)),
        ],
        out_specs=pl.BlockSpec((bn, d), lambda i: (i, 0)),
        out_shape=jax.ShapeDtypeStruct((n, d), jnp.float32),
    )(x, w_self, bias.reshape(1, d))


def _final_body(acc_ref, sp_ref, out_ref):
    t = acc_ref[0] + acc_ref[1] + sp_ref[...]
    out_ref[...] = jnp.maximum(t, 0.0)


def _final(acc, selfp, bn):
    n, d = selfp.shape
    return pl.pallas_call(
        _final_body,
        grid=(n // bn,),
        in_specs=[
            pl.BlockSpec((2, bn, d), lambda i: (0, i, 0)),
            pl.BlockSpec((bn, d), lambda i: (i, 0)),
        ],
        out_specs=pl.BlockSpec((bn, d), lambda i: (i, 0)),
        out_shape=jax.ShapeDtypeStruct((n, d), jnp.float32),
    )(acc, selfp)


def kernel(x, edge_index, edge_type, num_entities, weight,
           self_loop_weight, bias):
    n, d = x.shape
    r = weight.shape[0]
    e = edge_type.shape[0]
    src = edge_index[0]
    dst = edge_index[1]

    xw = _compute_xw(x, weight, bn=2000)
    xw_flat = xw.reshape(r * n, d)
    fidx = _compute_fidx(src, edge_type, n)

    nw = 32
    nt = e // nw
    sb_sz = 2000
    nsb = nt // sb_sz
    c_sz = 40
    nc = sb_sz // c_sz
    npad = ((n // 16 + 7) // 8 * 8) * 16
    zeros_blk = jnp.zeros((npad // 16, d), jnp.float32)

    acc = _sc_gather_scatter(xw_flat, fidx, dst, zeros_blk, npad, d,
                             c_sz, nt, sb_sz)
    selfp = _selfp(x, self_loop_weight, bias, bn=2000)
    return _final(acc, selfp, bn=1000)


# speedup vs baseline: 1.0342x; 1.0264x over previous
import functools

import jax
import jax.numpy as jnp
from jax import lax
from jax.experimental import pallas as pl
from jax.experimental.pallas import tpu as pltpu
from jax.experimental.pallas import tpu_sc as plsc


def _xw_body(x_ref, w_ref, out_ref):
    out_ref[0] = jnp.dot(x_ref[...], w_ref[0],
                         preferred_element_type=jnp.float32)


def _compute_xw(x, weight, bn):
    n, d = x.shape
    r = weight.shape[0]
    return pl.pallas_call(
        _xw_body,
        grid=(n // bn, r),
        in_specs=[
            pl.BlockSpec((bn, d), lambda i, ri: (i, 0)),
            pl.BlockSpec((1, d, d), lambda i, ri: (ri, 0, 0)),
        ],
        out_specs=pl.BlockSpec((1, bn, d), lambda i, ri: (ri, i, 0)),
        out_shape=jax.ShapeDtypeStruct((r, n, d), jnp.float32),
    )(x, weight)


def _sc_gather_scatter(xw_flat, ei_flat, edge_type, zeros_blk, npad,
                       d, c_sz, nt, sb_sz, n, e):
    nsb = nt // sb_sz
    nc = sb_sz // c_sz
    rows = npad // 16
    mesh = plsc.VectorSubcoreMesh(core_axis_name="c", subcore_axis_name="s")

    @functools.partial(
        pl.kernel,
        mesh=mesh,
        out_type=jax.ShapeDtypeStruct((2, npad, d), jnp.float32),
        scratch_types=[
            pltpu.VMEM((sb_sz,), jnp.int32),
            pltpu.VMEM((sb_sz,), jnp.int32),
            pltpu.VMEM((sb_sz,), jnp.int32),
            pltpu.VMEM((c_sz, d), jnp.float32),
            pltpu.VMEM((c_sz, d), jnp.float32),
            pltpu.VMEM_SHARED((npad, d), jnp.float32),
            pltpu.SemaphoreType.DMA,
            pltpu.SemaphoreType.DMA,
        ],
    )
    def sc_kernel(xw_hbm, ei_hbm, et_hbm, zeros_hbm, out_hbm,
                  fidx_v, et_v, dst_v, rows_a, rows_b, acc_sh, sem_a,
                  sem_b):
        core = lax.axis_index("c")
        sub = lax.axis_index("s")
        wid = core * 16 + sub
        pltpu.sync_copy(zeros_hbm, acc_sh.at[pl.ds(sub * rows, rows)])
        plsc.subcore_barrier()

        def start_gather(i, buf, sem):
            i = jnp.minimum(i, nc - 1)
            off = pl.multiple_of(i * c_sz, 8)
            return pltpu.async_copy(
                xw_hbm.at[fidx_v.at[pl.ds(off, c_sz)]], buf, sem)

        def wait_gather(buf, sem):
            pltpu.make_async_copy(
                xw_hbm.at[fidx_v.at[pl.ds(0, c_sz)]], buf, sem).wait()

        def superblock(sb, carry):
            base = pl.multiple_of(wid * nt + sb * sb_sz, 8)
            pltpu.sync_copy(ei_hbm.at[pl.ds(base, sb_sz)], fidx_v)
            pltpu.sync_copy(et_hbm.at[pl.ds(base, sb_sz)], et_v)
            pltpu.sync_copy(ei_hbm.at[pl.ds(e + base, sb_sz)], dst_v)

            def fx(g, cf):
                goff = pl.multiple_of(g * 16, 16)
                fidx_v[pl.ds(goff, 16)] = (et_v[pl.ds(goff, 16)] * n
                                           + fidx_v[pl.ds(goff, 16)])
                return cf

            lax.fori_loop(0, sb_sz // 16, fx, 0)
            start_gather(0, rows_a, sem_a)

            def dst_slice(i):
                off = pl.multiple_of(i * c_sz, 8)
                return dst_v.at[pl.ds(off, c_sz)]

            def pair(j, carry2):
                i0 = j * 2
                start_gather(i0 + 1, rows_b, sem_b)
                wait_gather(rows_a, sem_a)
                pltpu.sync_copy(rows_a, acc_sh.at[dst_slice(i0)], add=True)
                start_gather(i0 + 2, rows_a, sem_a)
                wait_gather(rows_b, sem_b)
                pltpu.sync_copy(rows_b, acc_sh.at[dst_slice(i0 + 1)],
                                add=True)
                return carry2

            lax.fori_loop(0, nc // 2, pair, 0)
            wait_gather(rows_a, sem_a)
            return carry

        lax.fori_loop(0, nsb, superblock, 0)
        plsc.subcore_barrier()
        pltpu.sync_copy(acc_sh.at[pl.ds(sub * rows, rows)],
                        out_hbm.at[core, pl.ds(sub * rows, rows)])

    return sc_kernel(xw_flat, ei_flat, edge_type, zeros_blk)


def _selfp_body(x_ref, w_ref, b_ref, out_ref):
    out_ref[...] = (jnp.dot(x_ref[...], w_ref[...],
                            preferred_element_type=jnp.float32)
                    + b_ref[...])


def _selfp(x, w_self, bias, bn):
    n, d = x.shape
    return pl.pallas_call(
        _selfp_body,
        grid=(n // bn,),
        in_specs=[
            pl.BlockSpec((bn, d), lambda i: (i, 0)),
            pl.BlockSpec((d, d), lambda i: (0, 0)),
            pl.BlockSpec((1, d), lambda i: (0, 0)),
        ],
        out_specs=pl.BlockSpec((bn, d), lambda i: (i, 0)),
        out_shape=jax.ShapeDtypeStruct((n, d), jnp.float32),
    )(x, w_self, bias.reshape(1, d))


def _final_body(acc_ref, sp_ref, out_ref):
    t = acc_ref[0] + acc_ref[1] + sp_ref[...]
    out_ref[...] = jnp.maximum(t, 0.0)


def _final(acc, selfp, bn):
    n, d = selfp.shape
    return pl.pallas_call(
        _final_body,
        grid=(n // bn,),
        in_specs=[
            pl.BlockSpec((2, bn, d), lambda i: (0, i, 0)),
            pl.BlockSpec((bn, d), lambda i: (i, 0)),
        ],
        out_specs=pl.BlockSpec((bn, d), lambda i: (i, 0)),
        out_shape=jax.ShapeDtypeStruct((n, d), jnp.float32),
    )(acc, selfp)


def kernel(x, edge_index, edge_type, num_entities, weight,
           self_loop_weight, bias):
    n, d = x.shape
    r = weight.shape[0]
    e = edge_type.shape[0]

    xw = _compute_xw(x, weight, bn=2000)
    xw_flat = xw.reshape(r * n, d)

    nw = 32
    nt = e // nw
    sb_sz = 2000
    nsb = nt // sb_sz
    c_sz = 40
    nc = sb_sz // c_sz
    npad = ((n // 16 + 7) // 8 * 8) * 16
    zeros_blk = jnp.zeros((npad // 16, d), jnp.float32)

    acc = _sc_gather_scatter(xw_flat, edge_index.reshape(2 * e),
                             edge_type, zeros_blk, npad, d, c_sz, nt,
                             sb_sz, n, e)
    selfp = _selfp(x, self_loop_weight, bias, bn=2000)
    return _final(acc, selfp, bn=1000)


# speedup vs baseline: 1.1369x; 1.0993x over previous
import functools

import jax
import jax.numpy as jnp
from jax import lax
from jax.experimental import pallas as pl
from jax.experimental.pallas import tpu as pltpu
from jax.experimental.pallas import tpu_sc as plsc


def _xw_body(x_ref, w_ref, out_ref):
    out_ref[0] = jnp.dot(x_ref[...], w_ref[0],
                         preferred_element_type=jnp.float32)


def _compute_xw(x, weight, bn):
    n, d = x.shape
    r = weight.shape[0]
    return pl.pallas_call(
        _xw_body,
        grid=(n // bn, r),
        in_specs=[
            pl.BlockSpec((bn, d), lambda i, ri: (i, 0)),
            pl.BlockSpec((1, d, d), lambda i, ri: (ri, 0, 0)),
        ],
        out_specs=pl.BlockSpec((1, bn, d), lambda i, ri: (ri, i, 0)),
        out_shape=jax.ShapeDtypeStruct((r, n, d), jnp.float32),
    )(x, weight)


def _sc_gather_scatter(xw_flat, ei_flat, edge_type, zeros_blk, npad,
                       d, c_sz, nt, sb_sz, n, e):
    nsb = nt // sb_sz
    nc = sb_sz // c_sz
    rows = npad // 16
    mesh = plsc.VectorSubcoreMesh(core_axis_name="c", subcore_axis_name="s")

    @functools.partial(
        pl.kernel,
        mesh=mesh,
        out_type=jax.ShapeDtypeStruct((2, npad, d), jnp.float32),
        scratch_types=[
            pltpu.VMEM((sb_sz,), jnp.int32),
            pltpu.VMEM((sb_sz,), jnp.int32),
            pltpu.VMEM((sb_sz,), jnp.int32),
            pltpu.VMEM((c_sz, d), jnp.float32),
            pltpu.VMEM((c_sz, d), jnp.float32),
            pltpu.VMEM_SHARED((npad, d), jnp.float32),
            pltpu.SemaphoreType.DMA,
            pltpu.SemaphoreType.DMA,
        ],
    )
    def sc_kernel(xw_hbm, ei_hbm, et_hbm, zeros_hbm, out_hbm,
                  fidx_v, et_v, dst_v, rows_a, rows_b, acc_sh, sem_a,
                  sem_b):
        core = lax.axis_index("c")
        sub = lax.axis_index("s")
        wid = core * 16 + sub
        pltpu.sync_copy(zeros_hbm, acc_sh.at[pl.ds(sub * rows, rows)])
        plsc.subcore_barrier()

        def start_gather(i, buf, sem):
            i = jnp.minimum(i, nc - 1)
            off = pl.multiple_of(i * c_sz, 8)
            return pltpu.async_copy(
                xw_hbm.at[fidx_v.at[pl.ds(off, c_sz)]], buf, sem)

        def wait_gather(buf, sem):
            pltpu.make_async_copy(
                xw_hbm.at[fidx_v.at[pl.ds(0, c_sz)]], buf, sem).wait()

        def superblock(sb, carry):
            base = pl.multiple_of(wid * nt + sb * sb_sz, 8)
            pltpu.sync_copy(ei_hbm.at[pl.ds(base, sb_sz)], fidx_v)
            pltpu.sync_copy(et_hbm.at[pl.ds(base, sb_sz)], et_v)
            pltpu.sync_copy(ei_hbm.at[pl.ds(e + base, sb_sz)], dst_v)

            def fx(g, cf):
                goff = pl.multiple_of(g * 16, 16)
                fidx_v[pl.ds(goff, 16)] = (et_v[pl.ds(goff, 16)] * n
                                           + fidx_v[pl.ds(goff, 16)])
                return cf

            lax.fori_loop(0, sb_sz // 16, fx, 0)
            start_gather(0, rows_a, sem_a)

            def dst_slice(i):
                off = pl.multiple_of(i * c_sz, 8)
                return dst_v.at[pl.ds(off, c_sz)]

            def pair(j, carry2):
                i0 = j * 2
                start_gather(i0 + 1, rows_b, sem_b)
                wait_gather(rows_a, sem_a)
                pltpu.sync_copy(rows_a, acc_sh.at[dst_slice(i0)], add=True)
                start_gather(i0 + 2, rows_a, sem_a)
                wait_gather(rows_b, sem_b)
                pltpu.sync_copy(rows_b, acc_sh.at[dst_slice(i0 + 1)],
                                add=True)
                return carry2

            lax.fori_loop(0, nc // 2, pair, 0)
            wait_gather(rows_a, sem_a)
            return carry

        lax.fori_loop(0, nsb, superblock, 0)
        plsc.subcore_barrier()
        pltpu.sync_copy(acc_sh.at[pl.ds(sub * rows, rows)],
                        out_hbm.at[core, pl.ds(sub * rows, rows)])

    return sc_kernel(xw_flat, ei_flat, edge_type, zeros_blk)


def _selfp_body(x_ref, w_ref, b_ref, out_ref):
    out_ref[...] = (jnp.dot(x_ref[...], w_ref[...],
                            preferred_element_type=jnp.float32)
                    + b_ref[...])


def _selfp(x, w_self, bias, bn):
    n, d = x.shape
    return pl.pallas_call(
        _selfp_body,
        grid=(n // bn,),
        in_specs=[
            pl.BlockSpec((bn, d), lambda i: (i, 0)),
            pl.BlockSpec((d, d), lambda i: (0, 0)),
            pl.BlockSpec((1, d), lambda i: (0, 0)),
        ],
        out_specs=pl.BlockSpec((bn, d), lambda i: (i, 0)),
        out_shape=jax.ShapeDtypeStruct((n, d), jnp.float32),
    )(x, w_self, bias.reshape(1, d))


def _final_body(acc_ref, sp_ref, out_ref):
    t = acc_ref[0] + acc_ref[1] + sp_ref[...]
    out_ref[...] = jnp.maximum(t, 0.0)


def _final(acc, selfp, bn):
    n, d = selfp.shape
    return pl.pallas_call(
        _final_body,
        grid=(n // bn,),
        in_specs=[
            pl.BlockSpec((2, bn, d), lambda i: (0, i, 0)),
            pl.BlockSpec((bn, d), lambda i: (i, 0)),
        ],
        out_specs=pl.BlockSpec((bn, d), lambda i: (i, 0)),
        out_shape=jax.ShapeDtypeStruct((n, d), jnp.float32),
    )(acc, selfp)


def kernel(x, edge_index, edge_type, num_entities, weight,
           self_loop_weight, bias):
    n, d = x.shape
    r = weight.shape[0]
    e = edge_type.shape[0]

    xw = _compute_xw(x, weight, bn=5000)
    xw_flat = xw.reshape(r * n, d)

    nw = 32
    nt = e // nw
    sb_sz = 2000
    nsb = nt // sb_sz
    c_sz = 40
    nc = sb_sz // c_sz
    npad = ((n // 16 + 7) // 8 * 8) * 16
    zeros_blk = jnp.zeros((npad // 16, d), jnp.float32)

    acc = _sc_gather_scatter(xw_flat, edge_index.reshape(2 * e),
                             edge_type, zeros_blk, npad, d, c_sz, nt,
                             sb_sz, n, e)
    selfp = _selfp(x, self_loop_weight, bias, bn=2000)
    return _final(acc, selfp, bn=2000)


# speedup vs baseline: 1.2033x; 1.0585x over previous
import functools

import jax
import jax.numpy as jnp
from jax import lax
from jax.experimental import pallas as pl
from jax.experimental.pallas import tpu as pltpu
from jax.experimental.pallas import tpu_sc as plsc


def _xw_body(x_ref, w_ref, out_ref):
    out_ref[0] = jnp.dot(x_ref[...], w_ref[0],
                         preferred_element_type=jnp.float32)


def _compute_xw(x, weight, bn):
    n, d = x.shape
    r = weight.shape[0]
    return pl.pallas_call(
        _xw_body,
        grid=(n // bn, r),
        in_specs=[
            pl.BlockSpec((bn, d), lambda i, ri: (i, 0)),
            pl.BlockSpec((1, d, d), lambda i, ri: (ri, 0, 0)),
        ],
        out_specs=pl.BlockSpec((1, bn, d), lambda i, ri: (ri, i, 0)),
        out_shape=jax.ShapeDtypeStruct((r, n, d), jnp.float32),
    )(x, weight)


def _sc_gather_scatter(xw_flat, ei_flat, edge_type, zeros_blk, npad,
                       d, c_sz, nt, sb_sz, n, e):
    nsb = nt // sb_sz
    nc = sb_sz // c_sz
    rows = npad // 16
    mesh = plsc.VectorSubcoreMesh(core_axis_name="c", subcore_axis_name="s")

    @functools.partial(
        pl.kernel,
        mesh=mesh,
        out_type=jax.ShapeDtypeStruct((2, npad, d), jnp.float32),
        scratch_types=[
            pltpu.VMEM((sb_sz,), jnp.int32),
            pltpu.VMEM((sb_sz,), jnp.int32),
            pltpu.VMEM((sb_sz,), jnp.int32),
            pltpu.VMEM((c_sz, d), jnp.float32),
            pltpu.VMEM((c_sz, d), jnp.float32),
            pltpu.VMEM_SHARED((npad, d), jnp.float32),
            pltpu.SemaphoreType.DMA,
            pltpu.SemaphoreType.DMA,
        ],
    )
    def sc_kernel(xw_hbm, ei_hbm, et_hbm, zeros_hbm, out_hbm,
                  fidx_v, et_v, dst_v, rows_a, rows_b, acc_sh, sem_a,
                  sem_b):
        core = lax.axis_index("c")
        sub = lax.axis_index("s")
        wid = core * 16 + sub
        pltpu.sync_copy(zeros_hbm, acc_sh.at[pl.ds(sub * rows, rows)])
        plsc.subcore_barrier()

        def start_gather(i, buf, sem):
            i = jnp.minimum(i, nc - 1)
            off = pl.multiple_of(i * c_sz, 8)
            return pltpu.async_copy(
                xw_hbm.at[fidx_v.at[pl.ds(off, c_sz)]], buf, sem)

        def wait_gather(buf, sem):
            pltpu.make_async_copy(
                xw_hbm.at[fidx_v.at[pl.ds(0, c_sz)]], buf, sem).wait()

        def superblock(sb, carry):
            base = pl.multiple_of(wid * nt + sb * sb_sz, 8)
            pltpu.sync_copy(ei_hbm.at[pl.ds(base, sb_sz)], fidx_v)
            pltpu.sync_copy(et_hbm.at[pl.ds(base, sb_sz)], et_v)
            pltpu.sync_copy(ei_hbm.at[pl.ds(e + base, sb_sz)], dst_v)

            def fx(g, cf):
                goff = pl.multiple_of(g * 16, 16)
                fidx_v[pl.ds(goff, 16)] = (et_v[pl.ds(goff, 16)] * n
                                           + fidx_v[pl.ds(goff, 16)])
                return cf

            lax.fori_loop(0, sb_sz // 16, fx, 0)
            start_gather(0, rows_a, sem_a)

            def dst_slice(i):
                off = pl.multiple_of(i * c_sz, 8)
                return dst_v.at[pl.ds(off, c_sz)]

            def pair(j, carry2):
                i0 = j * 2
                start_gather(i0 + 1, rows_b, sem_b)
                wait_gather(rows_a, sem_a)
                pltpu.sync_copy(rows_a, acc_sh.at[dst_slice(i0)], add=True)
                start_gather(i0 + 2, rows_a, sem_a)
                wait_gather(rows_b, sem_b)
                pltpu.sync_copy(rows_b, acc_sh.at[dst_slice(i0 + 1)],
                                add=True)
                return carry2

            lax.fori_loop(0, nc // 2, pair, 0)
            wait_gather(rows_a, sem_a)
            return carry

        lax.fori_loop(0, nsb, superblock, 0)
        plsc.subcore_barrier()
        pltpu.sync_copy(acc_sh.at[pl.ds(sub * rows, rows)],
                        out_hbm.at[core, pl.ds(sub * rows, rows)])

    return sc_kernel(xw_flat, ei_flat, edge_type, zeros_blk)


def _selfp_body(x_ref, w_ref, b_ref, out_ref):
    out_ref[...] = (jnp.dot(x_ref[...], w_ref[...],
                            preferred_element_type=jnp.float32)
                    + b_ref[...])


def _selfp(x, w_self, bias, bn):
    n, d = x.shape
    return pl.pallas_call(
        _selfp_body,
        grid=(n // bn,),
        in_specs=[
            pl.BlockSpec((bn, d), lambda i: (i, 0)),
            pl.BlockSpec((d, d), lambda i: (0, 0)),
            pl.BlockSpec((1, d), lambda i: (0, 0)),
        ],
        out_specs=pl.BlockSpec((bn, d), lambda i: (i, 0)),
        out_shape=jax.ShapeDtypeStruct((n, d), jnp.float32),
    )(x, w_self, bias.reshape(1, d))


def _final_body(acc_ref, sp_ref, out_ref):
    t = acc_ref[0] + acc_ref[1] + sp_ref[...]
    out_ref[...] = jnp.maximum(t, 0.0)


def _final(acc, selfp, bn):
    n, d = selfp.shape
    return pl.pallas_call(
        _final_body,
        grid=(n // bn,),
        in_specs=[
            pl.BlockSpec((2, bn, d), lambda i: (0, i, 0)),
            pl.BlockSpec((bn, d), lambda i: (i, 0)),
        ],
        out_specs=pl.BlockSpec((bn, d), lambda i: (i, 0)),
        out_shape=jax.ShapeDtypeStruct((n, d), jnp.float32),
    )(acc, selfp)


def kernel(x, edge_index, edge_type, num_entities, weight,
           self_loop_weight, bias):
    n, d = x.shape
    r = weight.shape[0]
    e = edge_type.shape[0]

    xw = _compute_xw(x, weight, bn=10000)
    xw_flat = xw.reshape(r * n, d)

    nw = 32
    nt = e // nw
    sb_sz = 2000
    nsb = nt // sb_sz
    c_sz = 40
    nc = sb_sz // c_sz
    npad = ((n // 16 + 7) // 8 * 8) * 16
    zeros_blk = jnp.zeros((npad // 16, d), jnp.float32)

    acc = _sc_gather_scatter(xw_flat, edge_index.reshape(2 * e),
                             edge_type, zeros_blk, npad, d, c_sz, nt,
                             sb_sz, n, e)
    selfp = _selfp(x, self_loop_weight, bias, bn=2000)
    return _final(acc, selfp, bn=2000)
